# baseline (speedup 1.0000x reference)
import math

import jax
import jax.numpy as jnp
from jax.experimental import pallas as pl
from jax.experimental.pallas import tpu as pltpu

_B = 64
_T = 10000
_NEG = float("-inf")
_NEG_F = -1e30
_BF = jnp.bfloat16
_F = jnp.float32


def _pass1(bcol_ref, brow_ref, x_ref, wa_ref,
           hsum_ref, cnt_ref, hmax_ref, smax_ref):
    i = pl.program_id(0)

    @pl.when(i == 0)
    def _():
        z = jnp.zeros_like(hsum_ref)
        hsum_ref[...] = z
        cnt_ref[...] = z
        hmax_ref[...] = jnp.full_like(hmax_ref, _NEG)
        smax_ref[...] = jnp.full_like(smax_ref, _NEG_F)

    x = x_ref[...]
    b = bcol_ref[0]
    brow = brow_ref[0]
    t = x.shape[0]
    d = x.shape[1]
    x16 = x.astype(_BF)
    st = jax.lax.dot_general(wa_ref[...].astype(_BF), x16,
                             (((1,), (1,)), ((), ())),
                             preferred_element_type=_F)
    seg = jax.lax.broadcasted_iota(jnp.int32, (_B, t), 0)
    ot16 = (seg == brow).astype(_BF)
    hsum_ref[...] += jnp.dot(ot16, x16, preferred_element_type=_F)

    lo = bcol_ref[0, 0, 0]
    hi = jnp.minimum(bcol_ref[0, t - 1, 0], _B - 1)

    nc = t // 8
    cmax = jnp.max(x.reshape(nc, 8, d), axis=1)
    cidx = jax.lax.broadcasted_iota(jnp.int32, (nc, 1), 0)
    lidx = jax.lax.broadcasted_iota(jnp.int32, (1, t), 1)

    def body(s, c):
        maskr = brow == s
        tm = jnp.max(jnp.where(maskr, st, _NEG), axis=1, keepdims=True)
        smax_ref[pl.ds(s, 1), :] = jnp.maximum(
            smax_ref[pl.ds(s, 1), :], jnp.broadcast_to(tm, (1, d)))
        c_s = jnp.sum(jnp.where(maskr, 1.0, 0.0), axis=1, keepdims=True)
        cnt_ref[pl.ds(s, 1), :] += jnp.broadcast_to(c_s, (1, d))
        r0 = jnp.min(jnp.where(maskr, lidx, t))
        r1 = jnp.max(jnp.where(maskr, lidx, -1))
        c0 = r0 // 8
        c1 = r1 // 8
        interior = (cidx > c0) & (cidx < c1)
        m = jnp.max(jnp.where(interior, cmax, _NEG), axis=0, keepdims=True)
        e0 = jnp.max(jnp.where(bcol_ref[0, pl.ds(c0 * 8, 8), :] == s,
                               x_ref[pl.ds(c0 * 8, 8), :], _NEG),
                     axis=0, keepdims=True)
        e1 = jnp.max(jnp.where(bcol_ref[0, pl.ds(c1 * 8, 8), :] == s,
                               x_ref[pl.ds(c1 * 8, 8), :], _NEG),
                     axis=0, keepdims=True)
        m = jnp.maximum(m, jnp.maximum(e0, e1))
        hmax_ref[pl.ds(s, 1), :] = jnp.maximum(hmax_ref[pl.ds(s, 1), :], m)
        return c

    jax.lax.fori_loop(lo, hi + 1, body, 0)


def _pass2(bcol_ref, brow_ref, x_ref, wk_ref, bk_ref, wq_ref, bq_ref,
           hsum_ref, cnt_ref, hmax_ref, smax_ref,
           wa_ref, w1t_ref, b1_ref, w2t_ref, b2_ref,
           out_ref,
           qt_scr, m2_scr, den2_scr, num2t_scr, numa_scr, dena_scr):
    i = pl.program_id(0)
    nt = pl.num_programs(0)
    rsd = 1.0 / math.sqrt(x_ref.shape[1])

    @pl.when(i == 0)
    def _():
        den2_scr[...] = jnp.zeros_like(den2_scr)
        num2t_scr[...] = jnp.zeros_like(num2t_scr)
        numa_scr[...] = jnp.zeros_like(numa_scr)
        dena_scr[...] = jnp.zeros_like(dena_scr)
        m2_scr[...] = jnp.full_like(m2_scr, _NEG_F)
        hmean = hsum_ref[...] / cnt_ref[...]
        qt_scr[...] = jax.lax.dot_general(
            wq_ref[...], hmean, (((1,), (1,)), ((), ())),
            preferred_element_type=_F) + bq_ref[...]

    x = x_ref[...]
    brow = brow_ref[0]
    t = x.shape[0]
    d = x.shape[1]
    lo = bcol_ref[0, 0, 0]
    hi = jnp.minimum(bcol_ref[0, t - 1, 0], _B - 1)
    x16 = x.astype(_BF)

    kt = jax.lax.dot_general(wk_ref[...].astype(_BF), x16,
                             (((1,), (1,)), ((), ())),
                             preferred_element_type=_F) + bk_ref[...]
    st = jax.lax.dot_general(wa_ref[...].astype(_BF), x16,
                             (((1,), (1,)), ((), ())),
                             preferred_element_type=_F)
    seg = jax.lax.broadcasted_iota(jnp.int32, (_B, t), 0)
    ot16 = (seg == brow).astype(_BF)
    qcols = jnp.dot(qt_scr[...].astype(_BF), ot16, preferred_element_type=_F)
    s2t = jnp.sum(kt * qcols, axis=0, keepdims=True) * rsd

    lane = jax.lax.broadcasted_iota(jnp.int32, (1, _B), 1)

    def body_a(s, carry):
        ea_t, e2_t = carry
        maskr = brow == s
        tm = jnp.max(jnp.where(maskr, s2t, _NEG), axis=1, keepdims=True)
        olds = m2_scr[pl.ds(s, 1), :]
        news = jnp.maximum(olds, jnp.broadcast_to(tm, (1, d)))
        scale = jnp.exp(olds - news)
        m2_scr[pl.ds(s, 1), :] = news
        fac = jnp.where(lane == s, scale[:, 0:1], 1.0)
        num2t_scr[...] = num2t_scr[...] * fac
        den2_scr[...] = den2_scr[...] * fac
        e2_s = jnp.exp(s2t - news[:, 0:1])
        den2_scr[...] += jnp.where(
            lane == s,
            jnp.sum(jnp.where(maskr, e2_s, 0.0), axis=1, keepdims=True), 0.0)
        sxs = smax_ref[pl.ds(s, 1), :][:, 0:1]
        ea_s = jnp.exp(st - sxs)
        dena_scr[pl.ds(s, 1), :] += jnp.broadcast_to(
            jnp.sum(jnp.where(maskr, ea_s, 0.0), axis=1, keepdims=True),
            (1, d))
        return (jnp.where(maskr, ea_s, ea_t), jnp.where(maskr, e2_s, e2_t))

    zrow = jnp.zeros((1, t), dtype=_F)
    ea_t, e2_t = jax.lax.fori_loop(lo, hi + 1, body_a, (zrow, zrow))

    ote_a = ot16 * ea_t.astype(_BF)
    ote_2 = ot16 * e2_t.astype(_BF)
    numa_scr[...] += jnp.dot(ote_a, x16, preferred_element_type=_F)
    num2t_scr[...] += jax.lax.dot_general(
        kt.astype(_BF), ote_2, (((1,), (1,)), ((), ())),
        preferred_element_type=_F)

    @pl.when(i == nt - 1)
    def _():
        hmean = hsum_ref[...] / cnt_ref[...]
        hattn = numa_scr[...] / dena_scr[...]
        eye = (jax.lax.broadcasted_iota(jnp.int32, (_B, _B), 0) ==
               jax.lax.broadcasted_iota(jnp.int32, (_B, _B), 1)).astype(_F)
        num2 = jax.lax.dot_general(eye, num2t_scr[...],
                                   (((1,), (1,)), ((), ())),
                                   preferred_element_type=_F)
        den2 = jax.lax.dot_general(eye, jnp.broadcast_to(den2_scr[...],
                                                         (d, _B)),
                                   (((1,), (1,)), ((), ())),
                                   preferred_element_type=_F)
        hs2s = num2 / den2
        comb = jnp.concatenate(
            [hmean, hmax_ref[...], hsum_ref[...], hattn, hs2s], axis=-1)
        h = jnp.dot(comb, w1t_ref[...],
                    preferred_element_type=_F) + b1_ref[...]
        h = 0.5 * h * (1.0 + jax.lax.erf(h * (1.0 / math.sqrt(2.0))))
        out_ref[...] = jnp.dot(h, w2t_ref[...],
                               preferred_element_type=_F) + b2_ref[...]


def kernel(x, batch, Wa, ba, Wq, bq, Wk, bk, W1, b1, W2, b2):
    n, d = x.shape
    t = _T
    nt = -(-n // t)
    npad = nt * t
    b32 = batch.astype(jnp.int32)
    if npad != n:
        x = jnp.pad(x, ((0, npad - n), (0, 0)))
        b32 = jnp.pad(b32, (0, npad - n), constant_values=_B)
    bcol = b32.reshape(nt, t, 1)
    brow = b32.reshape(nt, 1, t)

    w1t = W1.T
    w2t = W2.T
    wa2 = Wa.reshape(1, d)
    bkc = bk.reshape(d, 1)
    bqc = bq.reshape(d, 1)
    b12 = b1.reshape(1, b1.shape[0])
    b22 = b2.reshape(1, b2.shape[0])

    params = pltpu.CompilerParams(dimension_semantics=("arbitrary",))
    stat_shape = jax.ShapeDtypeStruct((_B, d), jnp.float32)
    stat_spec = pl.BlockSpec((_B, d), lambda i: (0, 0))
    col_spec = pl.BlockSpec((1, t, 1), lambda i: (i, 0, 0))
    row_spec = pl.BlockSpec((1, 1, t), lambda i: (i, 0, 0))
    x_spec = pl.BlockSpec((t, d), lambda i: (i, 0))

    hsum, cnt, hmax, smax = pl.pallas_call(
        _pass1,
        grid=(nt,),
        in_specs=[
            col_spec, row_spec, x_spec,
            pl.BlockSpec((1, d), lambda i: (0, 0)),
        ],
        out_specs=[stat_spec] * 4,
        out_shape=[stat_shape] * 4,
        compiler_params=params,
    )(bcol, brow, x, wa2)

    out = pl.pallas_call(
        _pass2,
        grid=(nt,),
        in_specs=[
            col_spec, row_spec, x_spec,
            pl.BlockSpec((d, d), lambda i: (0, 0)),
            pl.BlockSpec((d, 1), lambda i: (0, 0)),
            pl.BlockSpec((d, d), lambda i: (0, 0)),
            pl.BlockSpec((d, 1), lambda i: (0, 0)),
            stat_spec, stat_spec, stat_spec, stat_spec,
            pl.BlockSpec((1, d), lambda i: (0, 0)),
            pl.BlockSpec(w1t.shape, lambda i: (0, 0)),
            pl.BlockSpec(b12.shape, lambda i: (0, 0)),
            pl.BlockSpec(w2t.shape, lambda i: (0, 0)),
            pl.BlockSpec(b22.shape, lambda i: (0, 0)),
        ],
        out_specs=pl.BlockSpec((_B, d), lambda i: (0, 0)),
        out_shape=jax.ShapeDtypeStruct((_B, d), jnp.float32),
        scratch_shapes=[
            pltpu.VMEM((d, _B), jnp.float32),
            pltpu.VMEM((_B, d), jnp.float32),
            pltpu.VMEM((1, _B), jnp.float32),
            pltpu.VMEM((d, _B), jnp.float32),
            pltpu.VMEM((_B, d), jnp.float32),
            pltpu.VMEM((_B, d), jnp.float32),
        ],
        compiler_params=params,
    )(bcol, brow, x, Wk, bkc, Wq, bqc, hsum, cnt, hmax, smax,
      wa2, w1t, b12, w2t, b22)
    return out


# speedup vs baseline: 1.0851x; 1.0851x over previous
import math

import jax
import jax.numpy as jnp
from jax.experimental import pallas as pl
from jax.experimental.pallas import tpu as pltpu

_B = 64
_T = 4000
_NEG = float("-inf")
_NEG_F = -1e30
_BF = jnp.bfloat16
_F = jnp.float32


def _pass1(bcol_ref, brow_ref, x_ref, wa_ref,
           hsum_ref, cnt_ref, hmax_ref, smax_ref):
    i = pl.program_id(0)

    @pl.when(i == 0)
    def _():
        z = jnp.zeros_like(hsum_ref)
        hsum_ref[...] = z
        cnt_ref[...] = z
        hmax_ref[...] = jnp.full_like(hmax_ref, _NEG)
        smax_ref[...] = jnp.full_like(smax_ref, _NEG_F)

    x = x_ref[...]
    b = bcol_ref[0]
    brow = brow_ref[0]
    t = x.shape[0]
    d = x.shape[1]
    x16 = x.astype(_BF)
    st = jax.lax.dot_general(wa_ref[...].astype(_BF), x16,
                             (((1,), (1,)), ((), ())),
                             preferred_element_type=_F)
    seg = jax.lax.broadcasted_iota(jnp.int32, (_B, t), 0)
    ot16 = (seg == brow).astype(_BF)
    hsum_ref[...] += jnp.dot(ot16, x16, preferred_element_type=_F)

    lo = bcol_ref[0, 0, 0]
    hi = jnp.minimum(bcol_ref[0, t - 1, 0], _B - 1)

    nc = t // 8
    cmax = jnp.max(x.reshape(nc, 8, d), axis=1)
    cidx = jax.lax.broadcasted_iota(jnp.int32, (nc, 1), 0)
    lidx = jax.lax.broadcasted_iota(jnp.int32, (1, t), 1)

    def body(s, c):
        maskr = brow == s
        tm = jnp.max(jnp.where(maskr, st, _NEG), axis=1, keepdims=True)
        smax_ref[pl.ds(s, 1), :] = jnp.maximum(
            smax_ref[pl.ds(s, 1), :], jnp.broadcast_to(tm, (1, d)))
        c_s = jnp.sum(jnp.where(maskr, 1.0, 0.0), axis=1, keepdims=True)
        cnt_ref[pl.ds(s, 1), :] += jnp.broadcast_to(c_s, (1, d))
        r0 = jnp.min(jnp.where(maskr, lidx, t))
        r1 = jnp.max(jnp.where(maskr, lidx, -1))
        c0 = r0 // 8
        c1 = r1 // 8
        interior = (cidx > c0) & (cidx < c1)
        m = jnp.max(jnp.where(interior, cmax, _NEG), axis=0, keepdims=True)
        e0 = jnp.max(jnp.where(bcol_ref[0, pl.ds(c0 * 8, 8), :] == s,
                               x_ref[pl.ds(c0 * 8, 8), :], _NEG),
                     axis=0, keepdims=True)
        e1 = jnp.max(jnp.where(bcol_ref[0, pl.ds(c1 * 8, 8), :] == s,
                               x_ref[pl.ds(c1 * 8, 8), :], _NEG),
                     axis=0, keepdims=True)
        m = jnp.maximum(m, jnp.maximum(e0, e1))
        hmax_ref[pl.ds(s, 1), :] = jnp.maximum(hmax_ref[pl.ds(s, 1), :], m)
        return c

    jax.lax.fori_loop(lo, hi + 1, body, 0)


def _pass2(bcol_ref, brow_ref, x_ref, wk_ref, bk_ref, wq_ref, bq_ref,
           hsum_ref, cnt_ref, hmax_ref, smax_ref,
           wa_ref, w1t_ref, b1_ref, w2t_ref, b2_ref,
           out_ref,
           qt_scr, m2_scr, den2_scr, num2t_scr, numa_scr, dena_scr):
    i = pl.program_id(0)
    nt = pl.num_programs(0)
    rsd = 1.0 / math.sqrt(x_ref.shape[1])

    @pl.when(i == 0)
    def _():
        den2_scr[...] = jnp.zeros_like(den2_scr)
        num2t_scr[...] = jnp.zeros_like(num2t_scr)
        numa_scr[...] = jnp.zeros_like(numa_scr)
        dena_scr[...] = jnp.zeros_like(dena_scr)
        m2_scr[...] = jnp.full_like(m2_scr, _NEG_F)
        hmean = hsum_ref[...] / cnt_ref[...]
        qt_scr[...] = jax.lax.dot_general(
            wq_ref[...], hmean, (((1,), (1,)), ((), ())),
            preferred_element_type=_F) + bq_ref[...]

    x = x_ref[...]
    brow = brow_ref[0]
    t = x.shape[0]
    d = x.shape[1]
    lo = bcol_ref[0, 0, 0]
    hi = jnp.minimum(bcol_ref[0, t - 1, 0], _B - 1)
    x16 = x.astype(_BF)

    kt = jax.lax.dot_general(wk_ref[...].astype(_BF), x16,
                             (((1,), (1,)), ((), ())),
                             preferred_element_type=_F) + bk_ref[...]
    st = jax.lax.dot_general(wa_ref[...].astype(_BF), x16,
                             (((1,), (1,)), ((), ())),
                             preferred_element_type=_F)
    seg = jax.lax.broadcasted_iota(jnp.int32, (_B, t), 0)
    ot16 = (seg == brow).astype(_BF)
    qcols = jnp.dot(qt_scr[...].astype(_BF), ot16, preferred_element_type=_F)
    s2t = jnp.sum(kt * qcols, axis=0, keepdims=True) * rsd

    lane = jax.lax.broadcasted_iota(jnp.int32, (1, _B), 1)

    def body_a(s, carry):
        ea_t, e2_t = carry
        maskr = brow == s
        tm = jnp.max(jnp.where(maskr, s2t, _NEG), axis=1, keepdims=True)
        olds = m2_scr[pl.ds(s, 1), :]
        news = jnp.maximum(olds, jnp.broadcast_to(tm, (1, d)))
        scale = jnp.exp(olds - news)
        m2_scr[pl.ds(s, 1), :] = news
        fac = jnp.where(lane == s, scale[:, 0:1], 1.0)
        num2t_scr[...] = num2t_scr[...] * fac
        den2_scr[...] = den2_scr[...] * fac
        e2_s = jnp.exp(s2t - news[:, 0:1])
        den2_scr[...] += jnp.where(
            lane == s,
            jnp.sum(jnp.where(maskr, e2_s, 0.0), axis=1, keepdims=True), 0.0)
        sxs = smax_ref[pl.ds(s, 1), :][:, 0:1]
        ea_s = jnp.exp(st - sxs)
        dena_scr[pl.ds(s, 1), :] += jnp.broadcast_to(
            jnp.sum(jnp.where(maskr, ea_s, 0.0), axis=1, keepdims=True),
            (1, d))
        return (jnp.where(maskr, ea_s, ea_t), jnp.where(maskr, e2_s, e2_t))

    zrow = jnp.zeros((1, t), dtype=_F)
    ea_t, e2_t = jax.lax.fori_loop(lo, hi + 1, body_a, (zrow, zrow))

    ote_a = ot16 * ea_t.astype(_BF)
    ote_2 = ot16 * e2_t.astype(_BF)
    numa_scr[...] += jnp.dot(ote_a, x16, preferred_element_type=_F)
    num2t_scr[...] += jax.lax.dot_general(
        kt.astype(_BF), ote_2, (((1,), (1,)), ((), ())),
        preferred_element_type=_F)

    @pl.when(i == nt - 1)
    def _():
        hmean = hsum_ref[...] / cnt_ref[...]
        hattn = numa_scr[...] / dena_scr[...]
        eye = (jax.lax.broadcasted_iota(jnp.int32, (_B, _B), 0) ==
               jax.lax.broadcasted_iota(jnp.int32, (_B, _B), 1)).astype(_F)
        num2 = jax.lax.dot_general(eye, num2t_scr[...],
                                   (((1,), (1,)), ((), ())),
                                   preferred_element_type=_F)
        den2 = jax.lax.dot_general(eye, jnp.broadcast_to(den2_scr[...],
                                                         (d, _B)),
                                   (((1,), (1,)), ((), ())),
                                   preferred_element_type=_F)
        hs2s = num2 / den2
        comb = jnp.concatenate(
            [hmean, hmax_ref[...], hsum_ref[...], hattn, hs2s], axis=-1)
        h = jnp.dot(comb, w1t_ref[...],
                    preferred_element_type=_F) + b1_ref[...]
        h = 0.5 * h * (1.0 + jax.lax.erf(h * (1.0 / math.sqrt(2.0))))
        out_ref[...] = jnp.dot(h, w2t_ref[...],
                               preferred_element_type=_F) + b2_ref[...]


def kernel(x, batch, Wa, ba, Wq, bq, Wk, bk, W1, b1, W2, b2):
    n, d = x.shape
    t = _T
    nt = -(-n // t)
    npad = nt * t
    b32 = batch.astype(jnp.int32)
    if npad != n:
        x = jnp.pad(x, ((0, npad - n), (0, 0)))
        b32 = jnp.pad(b32, (0, npad - n), constant_values=_B)
    bcol = b32.reshape(nt, t, 1)
    brow = b32.reshape(nt, 1, t)

    w1t = W1.T
    w2t = W2.T
    wa2 = Wa.reshape(1, d)
    bkc = bk.reshape(d, 1)
    bqc = bq.reshape(d, 1)
    b12 = b1.reshape(1, b1.shape[0])
    b22 = b2.reshape(1, b2.shape[0])

    params = pltpu.CompilerParams(dimension_semantics=("arbitrary",))
    stat_shape = jax.ShapeDtypeStruct((_B, d), jnp.float32)
    stat_spec = pl.BlockSpec((_B, d), lambda i: (0, 0))
    col_spec = pl.BlockSpec((1, t, 1), lambda i: (i, 0, 0))
    row_spec = pl.BlockSpec((1, 1, t), lambda i: (i, 0, 0))
    x_spec = pl.BlockSpec((t, d), lambda i: (i, 0))

    hsum, cnt, hmax, smax = pl.pallas_call(
        _pass1,
        grid=(nt,),
        in_specs=[
            col_spec, row_spec, x_spec,
            pl.BlockSpec((1, d), lambda i: (0, 0)),
        ],
        out_specs=[stat_spec] * 4,
        out_shape=[stat_shape] * 4,
        compiler_params=params,
    )(bcol, brow, x, wa2)

    out = pl.pallas_call(
        _pass2,
        grid=(nt,),
        in_specs=[
            col_spec, row_spec, x_spec,
            pl.BlockSpec((d, d), lambda i: (0, 0)),
            pl.BlockSpec((d, 1), lambda i: (0, 0)),
            pl.BlockSpec((d, d), lambda i: (0, 0)),
            pl.BlockSpec((d, 1), lambda i: (0, 0)),
            stat_spec, stat_spec, stat_spec, stat_spec,
            pl.BlockSpec((1, d), lambda i: (0, 0)),
            pl.BlockSpec(w1t.shape, lambda i: (0, 0)),
            pl.BlockSpec(b12.shape, lambda i: (0, 0)),
            pl.BlockSpec(w2t.shape, lambda i: (0, 0)),
            pl.BlockSpec(b22.shape, lambda i: (0, 0)),
        ],
        out_specs=pl.BlockSpec((_B, d), lambda i: (0, 0)),
        out_shape=jax.ShapeDtypeStruct((_B, d), jnp.float32),
        scratch_shapes=[
            pltpu.VMEM((d, _B), jnp.float32),
            pltpu.VMEM((_B, d), jnp.float32),
            pltpu.VMEM((1, _B), jnp.float32),
            pltpu.VMEM((d, _B), jnp.float32),
            pltpu.VMEM((_B, d), jnp.float32),
            pltpu.VMEM((_B, d), jnp.float32),
        ],
        compiler_params=params,
    )(bcol, brow, x, Wk, bkc, Wq, bqc, hsum, cnt, hmax, smax,
      wa2, w1t, b12, w2t, b22)
    return out
